# baseline (speedup 1.0000x reference)
import functools

import jax
import jax.numpy as jnp
from jax import lax
from jax.experimental import pallas as pl
from jax.experimental.pallas import tpu as pltpu
from jax.experimental.pallas import tpu_sc as plsc

BATCH = 16384
EMBED_DIM = 64
PAIR = 2 * EMBED_DIM
NUM_CORES = 2
NUM_SUBCORES = 16
NUM_WORKERS = NUM_CORES * NUM_SUBCORES
BPW = BATCH // NUM_WORKERS
CHUNK = 128
NROUNDS = BPW // CHUNK
NBLK = CHUNK // 16
ENT_LINES = 500000
REL_LINES = 500


def _tec_body(h_hbm, r_hbm, t_hbm, ent2, rel2, out_hbm,
              hidxv, ridxv, tidxv, gidxh, gidxt, hpair, tpair, relb, outb,
              sem):
    wid = lax.axis_index("s") * NUM_CORES + lax.axis_index("c")
    base = wid * BPW

    pltpu.sync_copy(rel2, relb)
    pltpu.sync_copy(h_hbm.at[pl.ds(base, BPW)], hidxv)
    pltpu.sync_copy(r_hbm.at[pl.ds(base, BPW)], ridxv)
    pltpu.sync_copy(t_hbm.at[pl.ds(base, BPW)], tidxv)

    lanes = lax.iota(jnp.int32, 16)

    for rnd in range(NROUNDS):
        for i in range(NBLK):
            dst = pl.ds(i * 16, 16)
            src = pl.ds(rnd * CHUNK + i * 16, 16)
            gidxh[dst] = hidxv[src] >> 1
            gidxt[dst] = tidxv[src] >> 1
        ch = pltpu.async_copy(ent2.at[gidxh], hpair, sem)
        ct = pltpu.async_copy(ent2.at[gidxt], tpair, sem)
        ch.wait()
        ct.wait()

        def blk_body(kb, carry, rnd=rnd):
            src = pl.ds(rnd * CHUNK + kb * 16, 16)
            hv = hidxv[src]
            tv = tidxv[src]
            rv = ridxv[src]
            kv = kb * 16 + lanes
            offh = (hv & 1) * EMBED_DIM
            offt = (tv & 1) * EMBED_DIM
            rbase = rv * EMBED_DIM
            kout = kv * EMBED_DIM

            def d_body(d, c):
                vh = plsc.load_gather(hpair, [kv, offh + d])
                vt = plsc.load_gather(tpair, [kv, offt + d])
                ra = rbase + d
                vr = plsc.load_gather(relb, [ra >> 7, ra & 127])
                plsc.store_scatter(outb, [kout + d], vh + vr - vt)
                return c

            lax.fori_loop(0, EMBED_DIM, d_body, 0)
            return carry

        lax.fori_loop(0, NBLK, blk_body, 0)
        pltpu.sync_copy(
            outb,
            out_hbm.at[pl.ds((base + rnd * CHUNK) * EMBED_DIM,
                             CHUNK * EMBED_DIM)])


def kernel(h, r, t, entity_emb, relation_emb):
    ent2 = entity_emb.reshape(ENT_LINES, PAIR)
    rel2 = relation_emb.reshape(REL_LINES, PAIR)
    mesh = plsc.VectorSubcoreMesh(core_axis_name="c", subcore_axis_name="s")
    k = functools.partial(
        pl.kernel,
        mesh=mesh,
        compiler_params=pltpu.CompilerParams(
            use_tc_tiling_on_sc=False, needs_layout_passes=False),
        out_type=jax.ShapeDtypeStruct((BATCH * EMBED_DIM,), jnp.float32),
        scratch_types=[
            pltpu.VMEM((BPW,), jnp.int32),
            pltpu.VMEM((BPW,), jnp.int32),
            pltpu.VMEM((BPW,), jnp.int32),
            pltpu.VMEM((CHUNK,), jnp.int32),
            pltpu.VMEM((CHUNK,), jnp.int32),
            pltpu.VMEM((CHUNK, PAIR), jnp.float32),
            pltpu.VMEM((CHUNK, PAIR), jnp.float32),
            pltpu.VMEM((REL_LINES, PAIR), jnp.float32),
            pltpu.VMEM((CHUNK * EMBED_DIM,), jnp.float32),
            pltpu.SemaphoreType.DMA,
        ],
    )(_tec_body)
    out = k(h, r, t, ent2, rel2)
    return out.reshape(BATCH, EMBED_DIM)


# speedup vs baseline: 1.0014x; 1.0014x over previous
import functools

import jax
import jax.numpy as jnp
from jax import lax
from jax.experimental import pallas as pl
from jax.experimental.pallas import tpu as pltpu
from jax.experimental.pallas import tpu_sc as plsc

BATCH = 16384
EMBED_DIM = 64
PAIR = 2 * EMBED_DIM
NUM_CORES = 2
NUM_SUBCORES = 16
NUM_WORKERS = NUM_CORES * NUM_SUBCORES
BPW = BATCH // NUM_WORKERS
CHUNK = 128
NROUNDS = BPW // CHUNK
NBLK = CHUNK // 16
ENT_LINES = 500000
REL_LINES = 500


def _tec_body(h_hbm, r_hbm, t_hbm, ent2, rel2, out_hbm,
              hidxv, ridxv, tidxv, gidxh, gidxt, hpair, tpair, relb, outb,
              sem):
    wid = lax.axis_index("s") * NUM_CORES + lax.axis_index("c")
    base = wid * BPW

    pltpu.sync_copy(rel2, relb)
    pltpu.sync_copy(h_hbm.at[pl.ds(base, BPW)], hidxv)
    pltpu.sync_copy(r_hbm.at[pl.ds(base, BPW)], ridxv)
    pltpu.sync_copy(t_hbm.at[pl.ds(base, BPW)], tidxv)

    lanes = lax.iota(jnp.int32, 16)

    for rnd in range(NROUNDS):
        for i in range(NBLK):
            dst = pl.ds(i * 16, 16)
            src = pl.ds(rnd * CHUNK + i * 16, 16)
            gidxh[dst] = hidxv[src] >> 1
            gidxt[dst] = tidxv[src] >> 1
        ch = pltpu.async_copy(ent2.at[gidxh], hpair, sem)
        ct = pltpu.async_copy(ent2.at[gidxt], tpair, sem)
        ch.wait()
        ct.wait()

        def blk_body(kb, carry, rnd=rnd):
            src = pl.ds(rnd * CHUNK + kb * 16, 16)
            hv = hidxv[src]
            tv = tidxv[src]
            rv = ridxv[src]
            kv = kb * 16 + lanes
            offh = (hv & 1) * EMBED_DIM
            offt = (tv & 1) * EMBED_DIM
            rbase = rv * EMBED_DIM
            kout = kv * EMBED_DIM

            def d_body(d, c):
                vh = plsc.load_gather(hpair, [kv, offh + d])
                vt = plsc.load_gather(tpair, [kv, offt + d])
                ra = rbase + d
                vr = plsc.load_gather(relb, [ra >> 7, ra & 127])
                plsc.store_scatter(outb, [kout + d], vh + vr - vt)
                return c

            lax.fori_loop(0, EMBED_DIM, d_body, 0)
            return carry

        lax.fori_loop(0, NBLK, blk_body, 0)
        pltpu.sync_copy(
            outb,
            out_hbm.at[pl.ds((base + rnd * CHUNK) * EMBED_DIM,
                             CHUNK * EMBED_DIM)])


def kernel(h, r, t, entity_emb, relation_emb):
    ent2 = entity_emb.reshape(ENT_LINES, PAIR)
    rel2 = relation_emb.reshape(REL_LINES, PAIR)
    mesh = plsc.VectorSubcoreMesh(core_axis_name="c", subcore_axis_name="s")
    k = functools.partial(
        pl.kernel,
        mesh=mesh,
        compiler_params=pltpu.CompilerParams(
            use_tc_tiling_on_sc=True, needs_layout_passes=False),
        out_type=jax.ShapeDtypeStruct((BATCH * EMBED_DIM,), jnp.float32),
        scratch_types=[
            pltpu.VMEM((BPW,), jnp.int32),
            pltpu.VMEM((BPW,), jnp.int32),
            pltpu.VMEM((BPW,), jnp.int32),
            pltpu.VMEM((CHUNK,), jnp.int32),
            pltpu.VMEM((CHUNK,), jnp.int32),
            pltpu.VMEM((CHUNK, PAIR), jnp.float32),
            pltpu.VMEM((CHUNK, PAIR), jnp.float32),
            pltpu.VMEM((REL_LINES, PAIR), jnp.float32),
            pltpu.VMEM((CHUNK * EMBED_DIM,), jnp.float32),
            pltpu.SemaphoreType.DMA,
        ],
    )(_tec_body)
    out = k(h, r, t, ent2, rel2)
    return out.reshape(BATCH, EMBED_DIM)


# speedup vs baseline: 1.5496x; 1.5474x over previous
import functools

import jax
import jax.numpy as jnp
from jax import lax
from jax.experimental import pallas as pl
from jax.experimental.pallas import tpu as pltpu
from jax.experimental.pallas import tpu_sc as plsc

BATCH = 16384
EMBED_DIM = 64
GRP = 8
NUM_CORES = 2
NUM_SUBCORES = 16
NUM_WORKERS = NUM_CORES * NUM_SUBCORES
BPW = BATCH // NUM_WORKERS
BLK = 16
NBLKS = BPW // BLK
LANES = 16
SUBS = EMBED_DIM // LANES


def _tec_body(h_hbm, r_hbm, t_hbm, ent_hbm, rel_hbm, out_hbm,
              hidx, ridx, tidx, hsl, tsl, rsl, outb, sem):
    wid = lax.axis_index("s") * NUM_CORES + lax.axis_index("c")
    base = wid * BPW

    pltpu.sync_copy(h_hbm.at[pl.ds(base, BPW)], hidx)
    pltpu.sync_copy(r_hbm.at[pl.ds(base, BPW)], ridx)
    pltpu.sync_copy(t_hbm.at[pl.ds(base, BPW)], tidx)

    lanes = lax.iota(jnp.int32, LANES)

    def blk_body(blk, carry):
        src = pl.ds(blk * BLK, BLK)
        hv = hidx[src]
        tv = tidx[src]
        rv = ridx[src]
        ghv = hv >> 3
        gtv = tv >> 3
        grv = rv >> 3
        shv = hv & (GRP - 1)
        stv = tv & (GRP - 1)
        srv = rv & (GRP - 1)
        copies = []
        for j in range(BLK):
            m = lanes == j
            gh = jnp.max(jnp.where(m, ghv, 0))
            gt = jnp.max(jnp.where(m, gtv, 0))
            gr = jnp.max(jnp.where(m, grv, 0))
            copies.append(pltpu.async_copy(
                ent_hbm.at[pl.ds(pl.multiple_of(gh * GRP, GRP), GRP)],
                hsl.at[j], sem))
            copies.append(pltpu.async_copy(
                ent_hbm.at[pl.ds(pl.multiple_of(gt * GRP, GRP), GRP)],
                tsl.at[j], sem))
            copies.append(pltpu.async_copy(
                rel_hbm.at[pl.ds(pl.multiple_of(gr * GRP, GRP), GRP)],
                rsl.at[j], sem))
        for c in copies:
            c.wait()
        for j in range(BLK):
            m = lanes == j
            rh = jnp.max(jnp.where(m, shv, 0))
            rt = jnp.max(jnp.where(m, stv, 0))
            rr = jnp.max(jnp.where(m, srv, 0))
            row = blk * BLK + j
            for c in range(SUBS):
                s = pl.ds(c * LANES, LANES)
                outb[row, s] = hsl[j, rh, s] + rsl[j, rr, s] - tsl[j, rt, s]
        return carry

    lax.fori_loop(0, NBLKS, blk_body, 0)

    pltpu.sync_copy(outb, out_hbm.at[pl.ds(base, BPW)])


def kernel(h, r, t, entity_emb, relation_emb):
    mesh = plsc.VectorSubcoreMesh(core_axis_name="c", subcore_axis_name="s")
    k = functools.partial(
        pl.kernel,
        mesh=mesh,
        compiler_params=pltpu.CompilerParams(
            use_tc_tiling_on_sc=True, needs_layout_passes=False),
        out_type=jax.ShapeDtypeStruct((BATCH, EMBED_DIM), jnp.float32),
        scratch_types=[
            pltpu.VMEM((BPW,), jnp.int32),
            pltpu.VMEM((BPW,), jnp.int32),
            pltpu.VMEM((BPW,), jnp.int32),
            pltpu.VMEM((BLK, GRP, EMBED_DIM), jnp.float32),
            pltpu.VMEM((BLK, GRP, EMBED_DIM), jnp.float32),
            pltpu.VMEM((BLK, GRP, EMBED_DIM), jnp.float32),
            pltpu.VMEM((BPW, EMBED_DIM), jnp.float32),
            pltpu.SemaphoreType.DMA,
        ],
    )(_tec_body)
    return k(h, r, t, entity_emb, relation_emb)


# speedup vs baseline: 2.0861x; 1.3462x over previous
import functools

import jax
import jax.numpy as jnp
from jax import lax
from jax.experimental import pallas as pl
from jax.experimental.pallas import tpu as pltpu
from jax.experimental.pallas import tpu_sc as plsc

BATCH = 16384
EMBED_DIM = 64
GRP = 8
NUM_CORES = 2
NUM_SUBCORES = 16
NUM_WORKERS = NUM_CORES * NUM_SUBCORES
BPW = BATCH // NUM_WORKERS
BLK = 16
NBLKS = BPW // BLK
LANES = 16
SUBS = EMBED_DIM // LANES


def _tec_body(h_hbm, r_hbm, t_hbm, ent_hbm, rel_hbm, out_hbm,
              hidx, ridx, tidx, hsl, tsl, rsl, outb, sem):
    wid = lax.axis_index("s") * NUM_CORES + lax.axis_index("c")
    base = wid * BPW

    pltpu.sync_copy(h_hbm.at[pl.ds(base, BPW)], hidx)
    pltpu.sync_copy(r_hbm.at[pl.ds(base, BPW)], ridx)
    pltpu.sync_copy(t_hbm.at[pl.ds(base, BPW)], tidx)

    lanes = lax.iota(jnp.int32, LANES)

    def blk_body(blk, carry):
        src = pl.ds(blk * BLK, BLK)
        hv = hidx[src]
        tv = tidx[src]
        rv = ridx[src]
        ghv = hv >> 3
        gtv = tv >> 3
        grv = rv >> 3
        shv = hv & (GRP - 1)
        stv = tv & (GRP - 1)
        srv = rv & (GRP - 1)
        copies = []
        for j in range(BLK):
            m = lanes == j
            gh = jnp.max(jnp.where(m, ghv, 0))
            gt = jnp.max(jnp.where(m, gtv, 0))
            gr = jnp.max(jnp.where(m, grv, 0))
            copies.append(pltpu.async_copy(ent_hbm.at[gh], hsl.at[j], sem))
            copies.append(pltpu.async_copy(ent_hbm.at[gt], tsl.at[j], sem))
            copies.append(pltpu.async_copy(rel_hbm.at[gr], rsl.at[j], sem))
        for c in copies:
            c.wait()
        for j in range(BLK):
            m = lanes == j
            rh = jnp.max(jnp.where(m, shv, 0))
            rt = jnp.max(jnp.where(m, stv, 0))
            rr = jnp.max(jnp.where(m, srv, 0))
            row = blk * BLK + j
            for c in range(SUBS):
                s = pl.ds(c * LANES, LANES)
                outb[row, s] = hsl[j, rh, s] + rsl[j, rr, s] - tsl[j, rt, s]
        return carry

    lax.fori_loop(0, NBLKS, blk_body, 0)

    pltpu.sync_copy(outb, out_hbm.at[pl.ds(base, BPW)])


def kernel(h, r, t, entity_emb, relation_emb):
    ent3 = entity_emb.reshape(1000000 // GRP, GRP, EMBED_DIM)
    rel3 = relation_emb.reshape(1000 // GRP, GRP, EMBED_DIM)
    mesh = plsc.VectorSubcoreMesh(core_axis_name="c", subcore_axis_name="s")
    k = functools.partial(
        pl.kernel,
        mesh=mesh,
        compiler_params=pltpu.CompilerParams(
            use_tc_tiling_on_sc=True, needs_layout_passes=False),
        out_type=jax.ShapeDtypeStruct((BATCH, EMBED_DIM), jnp.float32),
        scratch_types=[
            pltpu.VMEM((BPW,), jnp.int32),
            pltpu.VMEM((BPW,), jnp.int32),
            pltpu.VMEM((BPW,), jnp.int32),
            pltpu.VMEM((BLK, GRP, EMBED_DIM), jnp.float32),
            pltpu.VMEM((BLK, GRP, EMBED_DIM), jnp.float32),
            pltpu.VMEM((BLK, GRP, EMBED_DIM), jnp.float32),
            pltpu.VMEM((BPW, EMBED_DIM), jnp.float32),
            pltpu.SemaphoreType.DMA,
        ],
    )(_tec_body)
    return k(h, r, t, ent3, rel3)


# speedup vs baseline: 2.2081x; 1.0585x over previous
import functools

import jax
import jax.numpy as jnp
from jax import lax
from jax.experimental import pallas as pl
from jax.experimental.pallas import tpu as pltpu
from jax.experimental.pallas import tpu_sc as plsc

BATCH = 16384
EMBED_DIM = 64
GRP = 8
NUM_CORES = 2
NUM_SUBCORES = 16
NUM_WORKERS = NUM_CORES * NUM_SUBCORES
BPW = BATCH // NUM_WORKERS
BLK = 16
NBLKS = BPW // BLK
LANES = 16
SUBS = EMBED_DIM // LANES


def _tec_body(h_hbm, r_hbm, t_hbm, ent_hbm, rel_hbm, out_hbm,
              hidx, ridx, tidx, hsl, tsl, rsl, outb, semA, semB):
    wid = lax.axis_index("s") * NUM_CORES + lax.axis_index("c")
    base = wid * BPW

    pltpu.sync_copy(h_hbm.at[pl.ds(base, BPW)], hidx)
    pltpu.sync_copy(r_hbm.at[pl.ds(base, BPW)], ridx)
    pltpu.sync_copy(t_hbm.at[pl.ds(base, BPW)], tidx)

    lanes = lax.iota(jnp.int32, LANES)

    def fire_blk(blk, buf, sem):
        src = pl.ds(blk * BLK, BLK)
        ghv = hidx[src] >> 3
        gtv = tidx[src] >> 3
        grv = ridx[src] >> 3
        for j in range(BLK):
            m = lanes == j
            gh = jnp.max(jnp.where(m, ghv, 0))
            gt = jnp.max(jnp.where(m, gtv, 0))
            gr = jnp.max(jnp.where(m, grv, 0))
            pltpu.async_copy(ent_hbm.at[gh], hsl.at[buf, j], sem)
            pltpu.async_copy(ent_hbm.at[gt], tsl.at[buf, j], sem)
            pltpu.async_copy(rel_hbm.at[gr], rsl.at[buf, j], sem)

    def compute_blk(blk, buf, sem):
        src = pl.ds(blk * BLK, BLK)
        shv = hidx[src] & (GRP - 1)
        stv = tidx[src] & (GRP - 1)
        srv = ridx[src] & (GRP - 1)
        pltpu.make_async_copy(ent_hbm.at[pl.ds(0, BLK)], hsl.at[buf], sem).wait()
        pltpu.make_async_copy(ent_hbm.at[pl.ds(0, BLK)], tsl.at[buf], sem).wait()
        pltpu.make_async_copy(ent_hbm.at[pl.ds(0, BLK)], rsl.at[buf], sem).wait()
        for j in range(BLK):
            m = lanes == j
            rh = jnp.max(jnp.where(m, shv, 0))
            rt = jnp.max(jnp.where(m, stv, 0))
            rr = jnp.max(jnp.where(m, srv, 0))
            for c in range(SUBS):
                s = pl.ds(c * LANES, LANES)
                outb[buf, j, s] = (hsl[buf, j, rh, s] + rsl[buf, j, rr, s]
                                   - tsl[buf, j, rt, s])
        pltpu.sync_copy(outb.at[buf],
                        out_hbm.at[pl.ds(base + blk * BLK, BLK)])

    fire_blk(0, 0, semA)

    def pipe_body(i, carry):
        fire_blk(2 * i + 1, 1, semB)
        compute_blk(2 * i, 0, semA)

        @pl.when(i < NBLKS // 2 - 1)
        def _():
            fire_blk(2 * i + 2, 0, semA)

        compute_blk(2 * i + 1, 1, semB)
        return carry

    lax.fori_loop(0, NBLKS // 2, pipe_body, 0)


def kernel(h, r, t, entity_emb, relation_emb):
    ent3 = entity_emb.reshape(1000000 // GRP, GRP, EMBED_DIM)
    rel3 = relation_emb.reshape(1000 // GRP, GRP, EMBED_DIM)
    mesh = plsc.VectorSubcoreMesh(core_axis_name="c", subcore_axis_name="s")
    k = functools.partial(
        pl.kernel,
        mesh=mesh,
        compiler_params=pltpu.CompilerParams(
            use_tc_tiling_on_sc=True, needs_layout_passes=False),
        out_type=jax.ShapeDtypeStruct((BATCH, EMBED_DIM), jnp.float32),
        scratch_types=[
            pltpu.VMEM((BPW,), jnp.int32),
            pltpu.VMEM((BPW,), jnp.int32),
            pltpu.VMEM((BPW,), jnp.int32),
            pltpu.VMEM((2, BLK, GRP, EMBED_DIM), jnp.float32),
            pltpu.VMEM((2, BLK, GRP, EMBED_DIM), jnp.float32),
            pltpu.VMEM((2, BLK, GRP, EMBED_DIM), jnp.float32),
            pltpu.VMEM((2, BLK, EMBED_DIM), jnp.float32),
            pltpu.SemaphoreType.DMA,
            pltpu.SemaphoreType.DMA,
        ],
    )(_tec_body)
    return k(h, r, t, ent3, rel3)


# speedup vs baseline: 2.2098x; 1.0008x over previous
import functools

import jax
import jax.numpy as jnp
from jax import lax
from jax.experimental import pallas as pl
from jax.experimental.pallas import tpu as pltpu
from jax.experimental.pallas import tpu_sc as plsc

BATCH = 16384
EMBED_DIM = 64
GRP = 8
NUM_CORES = 2
NUM_SUBCORES = 16
NUM_WORKERS = NUM_CORES * NUM_SUBCORES
BPW = BATCH // NUM_WORKERS
BLK = 16
NBLKS = BPW // BLK
LANES = 16
SUBS = EMBED_DIM // LANES


def _tec_body(h_hbm, r_hbm, t_hbm, ent_hbm, rel_hbm, out_hbm,
              hidx, ridx, tidx, hsl, tsl, rsl, outb, semA, semB, semO0, semO1):
    wid = lax.axis_index("s") * NUM_CORES + lax.axis_index("c")
    base = wid * BPW

    pltpu.sync_copy(h_hbm.at[pl.ds(base, BPW)], hidx)
    pltpu.sync_copy(r_hbm.at[pl.ds(base, BPW)], ridx)
    pltpu.sync_copy(t_hbm.at[pl.ds(base, BPW)], tidx)

    lanes = lax.iota(jnp.int32, LANES)

    def fire_blk(blk, buf, sem):
        src = pl.ds(blk * BLK, BLK)
        ghv = hidx[src] >> 3
        gtv = tidx[src] >> 3
        grv = ridx[src] >> 3
        for j in range(BLK):
            m = lanes == j
            gh = jnp.max(jnp.where(m, ghv, 0))
            gt = jnp.max(jnp.where(m, gtv, 0))
            gr = jnp.max(jnp.where(m, grv, 0))
            pltpu.async_copy(ent_hbm.at[gh], hsl.at[buf, j], sem)
            pltpu.async_copy(ent_hbm.at[gt], tsl.at[buf, j], sem)
            pltpu.async_copy(rel_hbm.at[gr], rsl.at[buf, j], sem)

    semO = (semO0, semO1)

    def compute_blk(blk, buf, sem, it):
        src = pl.ds(blk * BLK, BLK)
        shv = hidx[src] & (GRP - 1)
        stv = tidx[src] & (GRP - 1)
        srv = ridx[src] & (GRP - 1)
        pltpu.make_async_copy(ent_hbm.at[pl.ds(0, BLK)], hsl.at[buf], sem).wait()
        pltpu.make_async_copy(ent_hbm.at[pl.ds(0, BLK)], tsl.at[buf], sem).wait()
        pltpu.make_async_copy(ent_hbm.at[pl.ds(0, BLK)], rsl.at[buf], sem).wait()

        @pl.when(it > 0)
        def _():
            pltpu.make_async_copy(out_hbm.at[pl.ds(0, BLK)], outb.at[buf],
                                  semO[buf]).wait()

        bufv = jnp.broadcast_to(buf, (LANES,))
        for j in range(BLK):
            jv = jnp.broadcast_to(j, (LANES,))
            rhs = shv.at[jv].get(mode="promise_in_bounds")
            rts = stv.at[jv].get(mode="promise_in_bounds")
            rrs = srv.at[jv].get(mode="promise_in_bounds")
            for c in range(SUBS):
                cols = c * LANES + lanes
                vh = plsc.load_gather(hsl, [bufv, jv, rhs, cols])
                vt = plsc.load_gather(tsl, [bufv, jv, rts, cols])
                vr = plsc.load_gather(rsl, [bufv, jv, rrs, cols])
                outb[buf, j, pl.ds(c * LANES, LANES)] = vh + vr - vt
        pltpu.async_copy(outb.at[buf],
                         out_hbm.at[pl.ds(base + blk * BLK, BLK)], semO[buf])

    fire_blk(0, 0, semA)

    def pipe_body(i, carry):
        fire_blk(2 * i + 1, 1, semB)
        compute_blk(2 * i, 0, semA, i)

        @pl.when(i < NBLKS // 2 - 1)
        def _():
            fire_blk(2 * i + 2, 0, semA)

        compute_blk(2 * i + 1, 1, semB, i)
        return carry

    lax.fori_loop(0, NBLKS // 2, pipe_body, 0)

    pltpu.make_async_copy(out_hbm.at[pl.ds(0, BLK)], outb.at[0], semO0).wait()
    pltpu.make_async_copy(out_hbm.at[pl.ds(0, BLK)], outb.at[1], semO1).wait()


def kernel(h, r, t, entity_emb, relation_emb):
    ent3 = entity_emb.reshape(1000000 // GRP, GRP, EMBED_DIM)
    rel3 = relation_emb.reshape(1000 // GRP, GRP, EMBED_DIM)
    mesh = plsc.VectorSubcoreMesh(core_axis_name="c", subcore_axis_name="s")
    k = functools.partial(
        pl.kernel,
        mesh=mesh,
        compiler_params=pltpu.CompilerParams(
            use_tc_tiling_on_sc=True, needs_layout_passes=False),
        out_type=jax.ShapeDtypeStruct((BATCH, EMBED_DIM), jnp.float32),
        scratch_types=[
            pltpu.VMEM((BPW,), jnp.int32),
            pltpu.VMEM((BPW,), jnp.int32),
            pltpu.VMEM((BPW,), jnp.int32),
            pltpu.VMEM((2, BLK, GRP, EMBED_DIM), jnp.float32),
            pltpu.VMEM((2, BLK, GRP, EMBED_DIM), jnp.float32),
            pltpu.VMEM((2, BLK, GRP, EMBED_DIM), jnp.float32),
            pltpu.VMEM((2, BLK, EMBED_DIM), jnp.float32),
            pltpu.SemaphoreType.DMA,
            pltpu.SemaphoreType.DMA,
            pltpu.SemaphoreType.DMA,
            pltpu.SemaphoreType.DMA,
        ],
    )(_tec_body)
    return k(h, r, t, ent3, rel3)


# speedup vs baseline: 2.4039x; 1.0879x over previous
import functools

import jax
import jax.numpy as jnp
from jax import lax
from jax.experimental import pallas as pl
from jax.experimental.pallas import tpu as pltpu
from jax.experimental.pallas import tpu_sc as plsc

BATCH = 16384
EMBED_DIM = 64
GRP = 8
NUM_CORES = 2
NUM_SUBCORES = 16
NUM_WORKERS = NUM_CORES * NUM_SUBCORES
BPW = BATCH // NUM_WORKERS
BLK = 16
NBLKS = BPW // BLK
LANES = 16
SUBS = EMBED_DIM // LANES
PAIR = 2 * EMBED_DIM


def _tec_body(h_hbm, r_hbm, t_hbm, ent_hbm, rel_hbm, out_hbm,
              hidx, ridx, tidx, hsl, tsl, rlsl, rlidx, outb,
              semA, semB, semO0, semO1):
    wid = lax.axis_index("s") * NUM_CORES + lax.axis_index("c")
    base = wid * BPW

    pltpu.sync_copy(h_hbm.at[pl.ds(base, BPW)], hidx)
    pltpu.sync_copy(r_hbm.at[pl.ds(base, BPW)], ridx)
    pltpu.sync_copy(t_hbm.at[pl.ds(base, BPW)], tidx)

    lanes = lax.iota(jnp.int32, LANES)
    semO = (semO0, semO1)

    def fire_blk(blk, buf, sem):
        src = pl.ds(blk * BLK, BLK)
        ghv = hidx[src] >> 3
        gtv = tidx[src] >> 3
        rlidx[buf, :] = ridx[src] >> 1
        pltpu.async_copy(rel_hbm.at[rlidx.at[buf]], rlsl.at[buf], sem)
        for j in range(BLK):
            m = lanes == j
            gh = jnp.max(jnp.where(m, ghv, 0))
            gt = jnp.max(jnp.where(m, gtv, 0))
            pltpu.async_copy(ent_hbm.at[gh], hsl.at[buf, j], sem)
            pltpu.async_copy(ent_hbm.at[gt], tsl.at[buf, j], sem)

    def compute_blk(blk, buf, sem, it):
        src = pl.ds(blk * BLK, BLK)
        shv = hidx[src] & (GRP - 1)
        stv = tidx[src] & (GRP - 1)
        roffv = (ridx[src] & 1) * EMBED_DIM
        pltpu.make_async_copy(ent_hbm.at[pl.ds(0, BLK)], hsl.at[buf], sem).wait()
        pltpu.make_async_copy(ent_hbm.at[pl.ds(0, BLK)], tsl.at[buf], sem).wait()
        pltpu.make_async_copy(rel_hbm.at[pl.ds(0, BLK)], rlsl.at[buf], sem).wait()

        @pl.when(it > 0)
        def _():
            pltpu.make_async_copy(out_hbm.at[pl.ds(0, BLK)], outb.at[buf],
                                  semO[buf]).wait()

        bufv = jnp.broadcast_to(buf, (LANES,))
        for j in range(BLK):
            jv = jnp.broadcast_to(j, (LANES,))
            rhs = shv.at[jv].get(mode="promise_in_bounds")
            rts = stv.at[jv].get(mode="promise_in_bounds")
            ros = roffv.at[jv].get(mode="promise_in_bounds")
            for c in range(SUBS):
                cols = c * LANES + lanes
                vh = plsc.load_gather(hsl, [bufv, jv, rhs, cols])
                vt = plsc.load_gather(tsl, [bufv, jv, rts, cols])
                vr = plsc.load_gather(rlsl, [bufv, jv, ros + cols])
                outb[buf, j, pl.ds(c * LANES, LANES)] = vh + vr - vt
        pltpu.async_copy(outb.at[buf],
                         out_hbm.at[pl.ds(base + blk * BLK, BLK)], semO[buf])

    fire_blk(0, 0, semA)

    def pipe_body(i, carry):
        fire_blk(2 * i + 1, 1, semB)
        compute_blk(2 * i, 0, semA, i)

        @pl.when(i < NBLKS // 2 - 1)
        def _():
            fire_blk(2 * i + 2, 0, semA)

        compute_blk(2 * i + 1, 1, semB, i)
        return carry

    lax.fori_loop(0, NBLKS // 2, pipe_body, 0)

    pltpu.make_async_copy(out_hbm.at[pl.ds(0, BLK)], outb.at[0], semO0).wait()
    pltpu.make_async_copy(out_hbm.at[pl.ds(0, BLK)], outb.at[1], semO1).wait()


def kernel(h, r, t, entity_emb, relation_emb):
    ent3 = entity_emb.reshape(1000000 // GRP, GRP, EMBED_DIM)
    rel2 = relation_emb.reshape(500, PAIR)
    mesh = plsc.VectorSubcoreMesh(core_axis_name="c", subcore_axis_name="s")
    k = functools.partial(
        pl.kernel,
        mesh=mesh,
        compiler_params=pltpu.CompilerParams(
            use_tc_tiling_on_sc=True, needs_layout_passes=False),
        out_type=jax.ShapeDtypeStruct((BATCH, EMBED_DIM), jnp.float32),
        scratch_types=[
            pltpu.VMEM((BPW,), jnp.int32),
            pltpu.VMEM((BPW,), jnp.int32),
            pltpu.VMEM((BPW,), jnp.int32),
            pltpu.VMEM((2, BLK, GRP, EMBED_DIM), jnp.float32),
            pltpu.VMEM((2, BLK, GRP, EMBED_DIM), jnp.float32),
            pltpu.VMEM((2, BLK, PAIR), jnp.float32),
            pltpu.VMEM((2, BLK), jnp.int32),
            pltpu.VMEM((2, BLK, EMBED_DIM), jnp.float32),
            pltpu.SemaphoreType.DMA,
            pltpu.SemaphoreType.DMA,
            pltpu.SemaphoreType.DMA,
            pltpu.SemaphoreType.DMA,
        ],
    )(_tec_body)
    return k(h, r, t, ent3, rel2)
